# baseline (speedup 1.0000x reference)
import functools

import jax
import jax.numpy as jnp
from jax import lax
from jax.experimental import pallas as pl
from jax.experimental.pallas import tpu as pltpu

EPS = 1e-5
_VMEM_LIMIT = 48 << 20


def _stats_kernel(x_ref, sum_ref, sumsq_ref):
    @pl.when(pl.program_id(1) == 0)
    def _():
        sum_ref[...] = jnp.zeros_like(sum_ref)
        sumsq_ref[...] = jnp.zeros_like(sumsq_ref)

    x = x_ref[0]
    sum_ref[...] += jnp.sum(x, axis=(-2, -1), keepdims=True)
    sumsq_ref[...] += jnp.sum(x * x, axis=(-2, -1), keepdims=True)


def _norm_kernel(x_ref, sum_ref, sumsq_ref, gamma_ref, beta_ref, o_ref, *,
                 inv_count):
    mean = sum_ref[...] * inv_count
    var = jnp.maximum(sumsq_ref[...] * inv_count - mean * mean, 0.0)
    scale = gamma_ref[...] * lax.rsqrt(var + EPS)
    shift = beta_ref[...] - mean * scale
    o_ref[0] = x_ref[0] * scale + shift


@jax.jit
def _bn2d(x_nchw, gamma, beta):
    n, c, h, w = x_nchw.shape
    c_tile = c if c <= 128 else 128
    cb = c // c_tile

    sum_o, sumsq_o = pl.pallas_call(
        _stats_kernel,
        out_shape=(jax.ShapeDtypeStruct((c, 1, 1), jnp.float32),
                   jax.ShapeDtypeStruct((c, 1, 1), jnp.float32)),
        grid=(cb, n),
        in_specs=[pl.BlockSpec((1, c_tile, h, w),
                               lambda ci, ni: (ni, ci, 0, 0))],
        out_specs=(pl.BlockSpec((c_tile, 1, 1), lambda ci, ni: (ci, 0, 0)),
                   pl.BlockSpec((c_tile, 1, 1), lambda ci, ni: (ci, 0, 0))),
        compiler_params=pltpu.CompilerParams(
            dimension_semantics=("parallel", "arbitrary"),
            vmem_limit_bytes=_VMEM_LIMIT),
    )(x_nchw)

    y = pl.pallas_call(
        functools.partial(_norm_kernel, inv_count=1.0 / float(n * h * w)),
        out_shape=jax.ShapeDtypeStruct((n, c, h, w), x_nchw.dtype),
        grid=(n, cb),
        in_specs=[pl.BlockSpec((1, c_tile, h, w),
                               lambda ni, ci: (ni, ci, 0, 0)),
                  pl.BlockSpec((c_tile, 1, 1), lambda ni, ci: (ci, 0, 0)),
                  pl.BlockSpec((c_tile, 1, 1), lambda ni, ci: (ci, 0, 0)),
                  pl.BlockSpec((c_tile, 1, 1), lambda ni, ci: (ci, 0, 0)),
                  pl.BlockSpec((c_tile, 1, 1), lambda ni, ci: (ci, 0, 0))],
        out_specs=pl.BlockSpec((1, c_tile, h, w),
                               lambda ni, ci: (ni, ci, 0, 0)),
        compiler_params=pltpu.CompilerParams(
            dimension_semantics=("parallel", "parallel"),
            vmem_limit_bytes=_VMEM_LIMIT),
    )(x_nchw, sum_o, sumsq_o,
      gamma.astype(jnp.float32).reshape(c, 1, 1),
      beta.astype(jnp.float32).reshape(c, 1, 1))

    return y


def kernel(x_nchw, gamma, beta):
    return _bn2d(x_nchw, gamma, beta)


# speedup vs baseline: 5.3797x; 5.3797x over previous
import functools

import jax
import jax.numpy as jnp
from jax import lax
from jax.experimental import pallas as pl
from jax.experimental.pallas import tpu as pltpu

EPS = 1e-5
_VMEM_LIMIT = 48 << 20


def _stats_kernel(x_ref, sum_ref, sumsq_ref, *, steps):
    @pl.when(pl.program_id(1) == 0)
    def _():
        sum_ref[...] = jnp.zeros_like(sum_ref)
        sumsq_ref[...] = jnp.zeros_like(sumsq_ref)

    x = x_ref[0]
    sum_ref[0] += jnp.sum(x, axis=(0, 1), keepdims=False)[None]
    sumsq_ref[0] += jnp.sum(x * x, axis=(0, 1), keepdims=False)[None]


def _norm_kernel(x_ref, sum_ref, sumsq_ref, gamma_ref, beta_ref, o_ref, *,
                 inv_count):
    total = sum_ref[0] + sum_ref[1]
    total_sq = sumsq_ref[0] + sumsq_ref[1]
    mean = total * inv_count
    var = jnp.maximum(total_sq * inv_count - mean * mean, 0.0)
    scale = gamma_ref[...] * lax.rsqrt(var + EPS)
    shift = beta_ref[...] - mean * scale
    o_ref[0] = x_ref[0] * scale + shift


@jax.jit
def _bn2d(x_nchw, gamma, beta):
    n, c, h, w = x_nchw.shape
    x = jnp.transpose(x_nchw, (0, 2, 3, 1))
    half = n // 2

    sum_o, sumsq_o = pl.pallas_call(
        functools.partial(_stats_kernel, steps=half),
        out_shape=(jax.ShapeDtypeStruct((2, 1, c), jnp.float32),
                   jax.ShapeDtypeStruct((2, 1, c), jnp.float32)),
        grid=(2, half),
        in_specs=[pl.BlockSpec((1, h, w, c),
                               lambda nb, ni: (nb * half + ni, 0, 0, 0))],
        out_specs=(pl.BlockSpec((1, 1, c), lambda nb, ni: (nb, 0, 0)),
                   pl.BlockSpec((1, 1, c), lambda nb, ni: (nb, 0, 0))),
        compiler_params=pltpu.CompilerParams(
            dimension_semantics=("parallel", "arbitrary"),
            vmem_limit_bytes=_VMEM_LIMIT),
    )(x)

    y = pl.pallas_call(
        functools.partial(_norm_kernel, inv_count=1.0 / float(n * h * w)),
        out_shape=jax.ShapeDtypeStruct((n, h, w, c), x_nchw.dtype),
        grid=(n,),
        in_specs=[pl.BlockSpec((1, h, w, c), lambda ni: (ni, 0, 0, 0)),
                  pl.BlockSpec((2, 1, c), lambda ni: (0, 0, 0)),
                  pl.BlockSpec((2, 1, c), lambda ni: (0, 0, 0)),
                  pl.BlockSpec((1, c), lambda ni: (0, 0)),
                  pl.BlockSpec((1, c), lambda ni: (0, 0))],
        out_specs=pl.BlockSpec((1, h, w, c), lambda ni: (ni, 0, 0, 0)),
        compiler_params=pltpu.CompilerParams(
            dimension_semantics=("parallel",),
            vmem_limit_bytes=_VMEM_LIMIT),
    )(x, sum_o, sumsq_o,
      gamma.astype(jnp.float32).reshape(1, c),
      beta.astype(jnp.float32).reshape(1, c))

    return jnp.transpose(y, (0, 3, 1, 2))


def kernel(x_nchw, gamma, beta):
    return _bn2d(x_nchw, gamma, beta)


# speedup vs baseline: 5.7168x; 1.0627x over previous
import functools

import jax
import jax.numpy as jnp
from jax import lax
from jax.experimental import pallas as pl
from jax.experimental.pallas import tpu as pltpu

EPS = 1e-5
_VMEM_LIMIT = 48 << 20


def _stats_kernel(x_ref, sum_ref, sumsq_ref, *, steps):
    @pl.when(pl.program_id(1) == 0)
    def _():
        sum_ref[...] = jnp.zeros_like(sum_ref)
        sumsq_ref[...] = jnp.zeros_like(sumsq_ref)

    x = x_ref[...]
    sum_ref[0] += jnp.sum(x, axis=(0, 1, 2), keepdims=False)[None]
    sumsq_ref[0] += jnp.sum(x * x, axis=(0, 1, 2), keepdims=False)[None]


def _norm_kernel(x_ref, sum_ref, sumsq_ref, gamma_ref, beta_ref, o_ref, *,
                 inv_count):
    total = sum_ref[0] + sum_ref[1]
    total_sq = sumsq_ref[0] + sumsq_ref[1]
    mean = total * inv_count
    var = jnp.maximum(total_sq * inv_count - mean * mean, 0.0)
    scale = gamma_ref[...] * lax.rsqrt(var + EPS)
    shift = beta_ref[...] - mean * scale
    o_ref[...] = x_ref[...] * scale + shift


@jax.jit
def _bn2d(x_nchw, gamma, beta):
    n, c, h, w = x_nchw.shape
    x = jnp.transpose(x_nchw, (0, 2, 3, 1))
    half = n // 2
    sb = next(s for s in (4, 2, 1) if half % s == 0)
    nb_steps = half // sb
    vb = next(s for s in (2, 1) if n % s == 0)

    sum_o, sumsq_o = pl.pallas_call(
        functools.partial(_stats_kernel, steps=nb_steps),
        out_shape=(jax.ShapeDtypeStruct((2, 1, c), jnp.float32),
                   jax.ShapeDtypeStruct((2, 1, c), jnp.float32)),
        grid=(2, nb_steps),
        in_specs=[pl.BlockSpec((sb, h, w, c),
                               lambda nb, ni: (nb * nb_steps + ni, 0, 0, 0))],
        out_specs=(pl.BlockSpec((1, 1, c), lambda nb, ni: (nb, 0, 0)),
                   pl.BlockSpec((1, 1, c), lambda nb, ni: (nb, 0, 0))),
        compiler_params=pltpu.CompilerParams(
            dimension_semantics=("parallel", "arbitrary"),
            vmem_limit_bytes=_VMEM_LIMIT),
    )(x)

    y = pl.pallas_call(
        functools.partial(_norm_kernel, inv_count=1.0 / float(n * h * w)),
        out_shape=jax.ShapeDtypeStruct((n, h, w, c), x_nchw.dtype),
        grid=(n // vb,),
        in_specs=[pl.BlockSpec((vb, h, w, c), lambda ni: (ni, 0, 0, 0)),
                  pl.BlockSpec((2, 1, c), lambda ni: (0, 0, 0)),
                  pl.BlockSpec((2, 1, c), lambda ni: (0, 0, 0)),
                  pl.BlockSpec((1, c), lambda ni: (0, 0)),
                  pl.BlockSpec((1, c), lambda ni: (0, 0))],
        out_specs=pl.BlockSpec((vb, h, w, c), lambda ni: (ni, 0, 0, 0)),
        compiler_params=pltpu.CompilerParams(
            dimension_semantics=("parallel",),
            vmem_limit_bytes=_VMEM_LIMIT),
    )(x, sum_o, sumsq_o,
      gamma.astype(jnp.float32).reshape(1, c),
      beta.astype(jnp.float32).reshape(1, c))

    return jnp.transpose(y, (0, 3, 1, 2))


def kernel(x_nchw, gamma, beta):
    return _bn2d(x_nchw, gamma, beta)
